# baseline (speedup 1.0000x reference)
import jax
import jax.numpy as jnp
from jax.experimental import pallas as pl

NUM_TOKENS = 32768
HIDDEN = 768
N_EXPERTS = 64
TOP_K = 8
BLOCK_T = 8192


def _router_body(x_ref, w_ref, wout_ref, iout_ref):
    x = x_ref[...]
    w = w_ref[...]
    logits = jax.lax.dot_general(
        w, x, (((1,), (1,)), ((), ())), preferred_element_type=jnp.float32
    )
    s = jax.nn.sigmoid(logits)
    rows = jax.lax.broadcasted_iota(jnp.int32, s.shape, 0)
    vals = []
    idxs = []
    for k in range(TOP_K):
        m = jnp.max(s, axis=0)
        idx = jnp.argmax(s, axis=0)
        vals.append(m)
        idxs.append(idx)
        if k + 1 < TOP_K:
            s = jnp.where(rows == idx[None, :], -1.0, s)
    wv = jnp.stack(vals, axis=0)
    iv = jnp.stack(idxs, axis=0)
    denom = jnp.sum(wv, axis=0, keepdims=True) + 1e-20
    wout_ref[...] = wv / denom
    iout_ref[...] = iv


def kernel(hidden_states, gate_weight):
    n_blocks = NUM_TOKENS // BLOCK_T
    wv_t, iv_t = pl.pallas_call(
        _router_body,
        grid=(n_blocks,),
        in_specs=[
            pl.BlockSpec((BLOCK_T, HIDDEN), lambda i: (i, 0)),
            pl.BlockSpec((N_EXPERTS, HIDDEN), lambda i: (0, 0)),
        ],
        out_specs=[
            pl.BlockSpec((TOP_K, BLOCK_T), lambda i: (0, i)),
            pl.BlockSpec((TOP_K, BLOCK_T), lambda i: (0, i)),
        ],
        out_shape=[
            jax.ShapeDtypeStruct((TOP_K, NUM_TOKENS), jnp.float32),
            jax.ShapeDtypeStruct((TOP_K, NUM_TOKENS), jnp.int32),
        ],
    )(hidden_states, gate_weight)
    return wv_t.T, iv_t.T


# speedup vs baseline: 1.0390x; 1.0390x over previous
import jax
import jax.numpy as jnp
from jax.experimental import pallas as pl

NUM_TOKENS = 32768
HIDDEN = 768
N_EXPERTS = 64
TOP_K = 8
BLOCK_T = 4096


def _router_body(x_ref, w_ref, wout_ref, iout_ref):
    x = x_ref[...]
    w = w_ref[...]
    logits = jax.lax.dot_general(
        w, x, (((1,), (1,)), ((), ())), preferred_element_type=jnp.float32
    )
    s = jax.nn.sigmoid(logits)
    rows = jax.lax.broadcasted_iota(jnp.int32, s.shape, 0)
    vals = []
    idxs = []
    for k in range(TOP_K):
        m = jnp.max(s, axis=0)
        idx = jnp.argmax(s, axis=0)
        vals.append(m)
        idxs.append(idx)
        if k + 1 < TOP_K:
            s = jnp.where(rows == idx[None, :], -1.0, s)
    wv = jnp.stack(vals, axis=0)
    iv = jnp.stack(idxs, axis=0)
    denom = jnp.sum(wv, axis=0, keepdims=True) + 1e-20
    wout_ref[...] = wv / denom
    iout_ref[...] = iv


def kernel(hidden_states, gate_weight):
    n_blocks = NUM_TOKENS // BLOCK_T
    wv_t, iv_t = pl.pallas_call(
        _router_body,
        grid=(n_blocks,),
        in_specs=[
            pl.BlockSpec((BLOCK_T, HIDDEN), lambda i: (i, 0)),
            pl.BlockSpec((N_EXPERTS, HIDDEN), lambda i: (0, 0)),
        ],
        out_specs=[
            pl.BlockSpec((TOP_K, BLOCK_T), lambda i: (0, i)),
            pl.BlockSpec((TOP_K, BLOCK_T), lambda i: (0, i)),
        ],
        out_shape=[
            jax.ShapeDtypeStruct((TOP_K, NUM_TOKENS), jnp.float32),
            jax.ShapeDtypeStruct((TOP_K, NUM_TOKENS), jnp.int32),
        ],
    )(hidden_states, gate_weight)
    return wv_t.T, iv_t.T
